# baseline (speedup 1.0000x reference)
import functools

import jax
import jax.numpy as jnp
from jax import lax
from jax.experimental import pallas as pl
from jax.experimental.pallas import tpu as pltpu
from jax.experimental.pallas import tpu_sc as plsc

ENT_SIZE = 1000000
REL_SIZE = 1000
DIM = 64
B = 16384
MARGIN = 1.0

NC = 2
NS = 16
L = 16
NW = NC * NS
ROWS_PER_W = B // NW
CHUNK = 64
NCHUNK = ROWS_PER_W // CHUNK
GROUPS = CHUNK // L
W = 2 * DIM


def _sc_body(ent_ref, rel_ref,
             ih_ref, ir_ref, it_ref, jh_ref, jr_ref, jt_ref,
             ph_ref, pr_ref, pt_ref, qh_ref, qr_ref, qt_ref,
             loss_out, ns_out,
             bh0, br0, bt0, ch0, cr0, ct0,
             bh1, br1, bt1, ch1, cr1, ct1,
             vih, vir, vit, vjh, vjr, vjt,
             vph, vpr, vpt, vqh, vqr, vqt,
             ns_v, loss_v, sem0, sem1):
    wid = lax.axis_index("s") * NC + lax.axis_index("c")

    pltpu.sync_copy(ih_ref.at[wid], vih)
    pltpu.sync_copy(ir_ref.at[wid], vir)
    pltpu.sync_copy(it_ref.at[wid], vit)
    pltpu.sync_copy(jh_ref.at[wid], vjh)
    pltpu.sync_copy(jr_ref.at[wid], vjr)
    pltpu.sync_copy(jt_ref.at[wid], vjt)
    pltpu.sync_copy(ph_ref.at[wid], vph)
    pltpu.sync_copy(pr_ref.at[wid], vpr)
    pltpu.sync_copy(pt_ref.at[wid], vpt)
    pltpu.sync_copy(qh_ref.at[wid], vqh)
    pltpu.sync_copy(qr_ref.at[wid], vqr)
    pltpu.sync_copy(qt_ref.at[wid], vqt)

    bufsets = ((bh0, br0, bt0, ch0, cr0, ct0),
               (bh1, br1, bt1, ch1, cr1, ct1))
    pars = (vph, vpr, vpt, vqh, vqr, vqt)
    sems = (sem0, sem1)

    def fire(c, bufs, sem):
        bh, br, bt, ch, cr, ct = bufs
        return [
            pltpu.async_copy(ent_ref.at[vih.at[c]], bh, sem),
            pltpu.async_copy(rel_ref.at[vir.at[c]], br, sem),
            pltpu.async_copy(ent_ref.at[vit.at[c]], bt, sem),
            pltpu.async_copy(ent_ref.at[vjh.at[c]], ch, sem),
            pltpu.async_copy(rel_ref.at[vjr.at[c]], cr, sem),
            pltpu.async_copy(ent_ref.at[vjt.at[c]], ct, sem),
        ]

    iota = lax.iota(jnp.int32, L)

    def compute_chunk(c, bufs, ploss_acc):
        bh, br, bt, ch, cr, ct = bufs

        def group(g, acc):
            rowvec = g * L + iota
            sl = pl.ds(g * L, L)
            pav = [p[c, sl] for p in pars]

            def dstep(i, pn):
                p_acc, n_acc = pn
                dvec = (i & ~(L - 1)) + ((iota + i) & (L - 1))
                hp = plsc.load_gather(bh, [rowvec, dvec + pav[0]])
                rp = plsc.load_gather(br, [rowvec, dvec + pav[1]])
                tp = plsc.load_gather(bt, [rowvec, dvec + pav[2]])
                p_acc = p_acc + jnp.abs(hp + rp - tp)
                hn = plsc.load_gather(ch, [rowvec, dvec + pav[3]])
                rn = plsc.load_gather(cr, [rowvec, dvec + pav[4]])
                tn = plsc.load_gather(ct, [rowvec, dvec + pav[5]])
                n_acc = n_acc + jnp.abs(hn + rn - tn)
                return (p_acc, n_acc)

            zero = jnp.zeros((L,), jnp.float32)
            p_acc, n_acc = lax.fori_loop(0, DIM, dstep, (zero, zero))
            ns_v[pl.ds(c * CHUNK + g * L, L)] = -n_acc
            return acc + jnp.maximum(p_acc - n_acc + MARGIN, 0.0)

        return lax.fori_loop(0, GROUPS, group, ploss_acc)

    ploss = jnp.zeros((L,), jnp.float32)
    descs = fire(0, bufsets[0], sems[0])
    for c in range(NCHUNK):
        nxt = fire(c + 1, bufsets[(c + 1) % 2], sems[(c + 1) % 2]) \
            if c + 1 < NCHUNK else None
        for d in descs:
            d.wait()
        ploss = compute_chunk(c, bufsets[c % 2], ploss)
        descs = nxt

    loss_v[...] = ploss
    pltpu.sync_copy(loss_v, loss_out.at[wid])
    pltpu.sync_copy(ns_v, ns_out.at[pl.ds(wid * ROWS_PER_W, ROWS_PER_W)])


@jax.jit
def _sc_call(ent2, rel2, *idx):
    mesh = plsc.VectorSubcoreMesh(core_axis_name="c", subcore_axis_name="s",
                                  num_cores=NC, num_subcores=NS)
    f = pl.kernel(
        _sc_body,
        out_type=(jax.ShapeDtypeStruct((NW, L), jnp.float32),
                  jax.ShapeDtypeStruct((B,), jnp.float32)),
        mesh=mesh,
        scratch_types=(
            [pltpu.VMEM((CHUNK, W), jnp.float32) for _ in range(12)]
            + [pltpu.VMEM((NCHUNK, CHUNK), jnp.int32) for _ in range(12)]
            + [pltpu.VMEM((ROWS_PER_W,), jnp.float32),
               pltpu.VMEM((L,), jnp.float32),
               pltpu.SemaphoreType.DMA,
               pltpu.SemaphoreType.DMA]
        ),
        compiler_params=pltpu.CompilerParams(needs_layout_passes=False,
                                             use_tc_tiling_on_sc=False),
    )
    return f(ent2, rel2, *idx)


def kernel(pos_h, pos_r, pos_t, neg_h, neg_r, neg_t, take, ent_emb, rel_emb):
    ent2 = ent_emb.reshape(ENT_SIZE // 2, W)
    rel2 = rel_emb.reshape(REL_SIZE // 2, W)
    shp = (NW, NCHUNK, CHUNK)
    halves = []
    parities = []
    for a in (pos_h, pos_r, pos_t, neg_h, neg_r, neg_t):
        a = a.astype(jnp.int32)
        halves.append((a >> 1).reshape(shp))
        parities.append(((a & 1) * DIM).reshape(shp))
    partials, neg_ns = _sc_call(ent2, rel2, *halves, *parities)
    return (jnp.sum(partials), neg_ns)


# speedup vs baseline: 1.0313x; 1.0313x over previous
import functools

import jax
import jax.numpy as jnp
from jax import lax
from jax.experimental import pallas as pl
from jax.experimental.pallas import tpu as pltpu
from jax.experimental.pallas import tpu_sc as plsc

ENT_SIZE = 1000000
REL_SIZE = 1000
DIM = 64
B = 16384
MARGIN = 1.0

NC = 2
NS = 16
L = 16
NW = NC * NS
ROWS_PER_W = B // NW
CHUNK = 64
NCHUNK = ROWS_PER_W // CHUNK
GROUPS = CHUNK // L


def _sc_body(ent_ref, rel_ref,
             ih_ref, ir_ref, it_ref, jh_ref, jr_ref, jt_ref,
             loss_out, ns_out,
             bh0, br0, bt0, ch0, cr0, ct0,
             bh1, br1, bt1, ch1, cr1, ct1,
             vih, vir, vit, vjh, vjr, vjt,
             ns_v, loss_v, sem0, sem1):
    wid = lax.axis_index("s") * NC + lax.axis_index("c")

    pltpu.sync_copy(ih_ref.at[wid], vih)
    pltpu.sync_copy(ir_ref.at[wid], vir)
    pltpu.sync_copy(it_ref.at[wid], vit)
    pltpu.sync_copy(jh_ref.at[wid], vjh)
    pltpu.sync_copy(jr_ref.at[wid], vjr)
    pltpu.sync_copy(jt_ref.at[wid], vjt)

    bufsets = ((bh0, br0, bt0, ch0, cr0, ct0),
               (bh1, br1, bt1, ch1, cr1, ct1))
    sems = (sem0, sem1)

    def fire(c, bufs, sem):
        bh, br, bt, ch, cr, ct = bufs
        return [
            pltpu.async_copy(ent_ref.at[vih.at[c]], bh, sem),
            pltpu.async_copy(rel_ref.at[vir.at[c]], br, sem),
            pltpu.async_copy(ent_ref.at[vit.at[c]], bt, sem),
            pltpu.async_copy(ent_ref.at[vjh.at[c]], ch, sem),
            pltpu.async_copy(rel_ref.at[vjr.at[c]], cr, sem),
            pltpu.async_copy(ent_ref.at[vjt.at[c]], ct, sem),
        ]

    iota = lax.iota(jnp.int32, L)

    def compute_chunk(c, bufs, ploss_acc):
        bh, br, bt, ch, cr, ct = bufs

        def group(g, acc):
            rowvec = g * L + iota

            def dstep(i, pn):
                p_acc, n_acc = pn
                dvec = (i & ~(L - 1)) + ((iota + i) & (L - 1))
                hp = plsc.load_gather(bh, [rowvec, dvec])
                rp = plsc.load_gather(br, [rowvec, dvec])
                tp = plsc.load_gather(bt, [rowvec, dvec])
                p_acc = p_acc + jnp.abs(hp + rp - tp)
                hn = plsc.load_gather(ch, [rowvec, dvec])
                rn = plsc.load_gather(cr, [rowvec, dvec])
                tn = plsc.load_gather(ct, [rowvec, dvec])
                n_acc = n_acc + jnp.abs(hn + rn - tn)
                return (p_acc, n_acc)

            zero = jnp.zeros((L,), jnp.float32)
            p_acc, n_acc = lax.fori_loop(0, DIM, dstep, (zero, zero))
            ns_v[pl.ds(c * CHUNK + g * L, L)] = -n_acc
            return acc + jnp.maximum(p_acc - n_acc + MARGIN, 0.0)

        return lax.fori_loop(0, GROUPS, group, ploss_acc)

    ploss = jnp.zeros((L,), jnp.float32)
    descs = fire(0, bufsets[0], sems[0])
    for c in range(NCHUNK):
        nxt = fire(c + 1, bufsets[(c + 1) % 2], sems[(c + 1) % 2]) \
            if c + 1 < NCHUNK else None
        for d in descs:
            d.wait()
        ploss = compute_chunk(c, bufsets[c % 2], ploss)
        descs = nxt

    loss_v[...] = ploss
    pltpu.sync_copy(loss_v, loss_out.at[wid])
    pltpu.sync_copy(ns_v, ns_out.at[pl.ds(wid * ROWS_PER_W, ROWS_PER_W)])


@jax.jit
def _sc_call(ent, rel, *idx):
    mesh = plsc.VectorSubcoreMesh(core_axis_name="c", subcore_axis_name="s",
                                  num_cores=NC, num_subcores=NS)
    f = pl.kernel(
        _sc_body,
        out_type=(jax.ShapeDtypeStruct((NW, L), jnp.float32),
                  jax.ShapeDtypeStruct((B,), jnp.float32)),
        mesh=mesh,
        scratch_types=(
            [pltpu.VMEM((CHUNK, DIM), jnp.float32) for _ in range(12)]
            + [pltpu.VMEM((NCHUNK, CHUNK), jnp.int32) for _ in range(6)]
            + [pltpu.VMEM((ROWS_PER_W,), jnp.float32),
               pltpu.VMEM((L,), jnp.float32),
               pltpu.SemaphoreType.DMA,
               pltpu.SemaphoreType.DMA]
        ),
        compiler_params=pltpu.CompilerParams(needs_layout_passes=False,
                                             use_tc_tiling_on_sc=False),
    )
    return f(ent, rel, *idx)


def kernel(pos_h, pos_r, pos_t, neg_h, neg_r, neg_t, take, ent_emb, rel_emb):
    shp = (NW, NCHUNK, CHUNK)
    idx = [a.astype(jnp.int32).reshape(shp)
           for a in (pos_h, pos_r, pos_t, neg_h, neg_r, neg_t)]
    partials, neg_ns = _sc_call(ent_emb, rel_emb, *idx)
    return (jnp.sum(partials), neg_ns)
